# Initial kernel scaffold; baseline (speedup 1.0000x reference)
import jax
import jax.numpy as jnp
from jax.experimental import pallas as pl


def kernel(inputs, W, usage_counts):
    raise NotImplementedError("write your pallas kernel here")


# speedup vs baseline: 1.6613x; 1.6613x over previous
import functools

import jax
import jax.numpy as jnp
import numpy as np
from jax import lax
from jax.experimental import pallas as pl
from jax.experimental.pallas import tpu as pltpu
from jax.experimental.pallas import tpu_sc as plsc

_NUM_CODES = 128
_CODE_DIM = 48
_COMMIT_W = 0.25
_EPS = 1e-05
_ENT_W = 0.1
_ENT_LO = 0.5
_ENT_HI = 0.9
_VAR_FLOOR = 0.05
_VAR_W = 0.001
_DECOR_W = 0.001

_N_ROWS = 128 * 1024
_BLK = 2048
_N_BLOCKS = _N_ROWS // _BLK


def _tc_body(x_ref, w_ref, u_ref, idx_ref, tot_ref, com_ref, ent_ref,
             var_ref, dec_ref, ue_ref, acc_ref):
    i = pl.program_id(0)
    x = x_ref[0]
    w = w_ref[...]

    xsq = jnp.sum(x * x, axis=1, keepdims=True)
    wsq = jnp.sum(w * w, axis=1)
    mm = jax.lax.dot_general(x, w, (((1,), (1,)), ((), ())),
                             preferred_element_type=jnp.float32)
    d = xsq - 2.0 * mm + wsq[None, :]

    mind = jnp.min(d, axis=1, keepdims=True)
    code_iota = lax.broadcasted_iota(jnp.int32, d.shape, 1)
    idx = jnp.min(jnp.where(d == mind, code_iota, _NUM_CODES), axis=1)
    idx_ref[0, 0, :] = idx

    blk_sum = jnp.sum(mind)

    @pl.when(i == 0)
    def _init():
        acc_ref[0] = blk_sum

    @pl.when(i > 0)
    def _acc():
        acc_ref[0] = acc_ref[0] + blk_sum

    @pl.when(i == _N_BLOCKS - 1)
    def _finalize():
        commit = _COMMIT_W * acc_ref[0] / float(_N_ROWS * _CODE_DIM)

        u = u_ref[...]
        p = u + _EPS
        p = p / jnp.maximum(jnp.sum(p), _EPS * _NUM_CODES)
        entropy = -jnp.sum(p * jnp.log(p + _EPS))
        ue = entropy / np.log(float(_NUM_CODES))
        gap = jnp.where(ue < _ENT_LO, _ENT_LO - ue,
                        jnp.where(ue > _ENT_HI, ue - _ENT_HI, 0.0))
        ent_loss = _ENT_W * gap * gap

        mean_w = jnp.mean(w, axis=0, keepdims=True)
        wc = w - mean_w
        variance = jnp.mean(wc * wc, axis=0, keepdims=True)
        var_loss = _VAR_W * jnp.mean(jnp.maximum(_VAR_FLOOR - variance, 0.0))

        cov = jax.lax.dot_general(wc, wc, (((0,), (0,)), ((), ())),
                                  preferred_element_type=jnp.float32)
        cov = cov / float(_NUM_CODES)
        ii = lax.broadcasted_iota(jnp.int32, cov.shape, 0)
        jj = lax.broadcasted_iota(jnp.int32, cov.shape, 1)
        off = jnp.where(ii == jj, 0.0, cov)
        dec_loss = _DECOR_W * jnp.sum(off * off) / float(_CODE_DIM * _CODE_DIM)

        tot_ref[...] = jnp.reshape(commit + ent_loss + var_loss + dec_loss,
                                   (1, 1))
        com_ref[...] = jnp.reshape(commit, (1, 1))
        ent_ref[...] = jnp.reshape(ent_loss, (1, 1))
        var_ref[...] = jnp.reshape(var_loss, (1, 1))
        dec_ref[...] = jnp.reshape(dec_loss, (1, 1))
        ue_ref[...] = jnp.reshape(ue, (1, 1))


def _tc_search(flat3, w, u2, interpret=False):
    scal = jax.ShapeDtypeStruct((1, 1), jnp.float32)
    return pl.pallas_call(
        _tc_body,
        grid=(_N_BLOCKS,),
        in_specs=[
            pl.BlockSpec((1, _BLK, _CODE_DIM), lambda i: (i, 0, 0)),
            pl.BlockSpec((_NUM_CODES, _CODE_DIM), lambda i: (0, 0)),
            pl.BlockSpec((1, _NUM_CODES), lambda i: (0, 0)),
        ],
        out_specs=[
            pl.BlockSpec((1, 1, _BLK), lambda i: (i, 0, 0)),
            pl.BlockSpec((1, 1), lambda i: (0, 0)),
            pl.BlockSpec((1, 1), lambda i: (0, 0)),
            pl.BlockSpec((1, 1), lambda i: (0, 0)),
            pl.BlockSpec((1, 1), lambda i: (0, 0)),
            pl.BlockSpec((1, 1), lambda i: (0, 0)),
            pl.BlockSpec((1, 1), lambda i: (0, 0)),
        ],
        out_shape=[
            jax.ShapeDtypeStruct((_N_BLOCKS, 1, _BLK), jnp.int32),
            scal, scal, scal, scal, scal, scal,
        ],
        scratch_shapes=[pltpu.SMEM((1,), jnp.float32)],
        interpret=interpret,
    )(flat3, w, u2)


_NW = 32
_B_PER_W = _N_ROWS // _NW
_CH = 512
_N_CH = _B_PER_W // _CH


def _sc_gather_body(w_hbm, idx_hbm, out_hbm, idx_v, rows_v, sem):
    wid = lax.axis_index("s") * 2 + lax.axis_index("c")
    base0 = wid * _B_PER_W

    def chunk(c, carry):
        base = base0 + c * _CH
        pltpu.sync_copy(idx_hbm.at[pl.ds(base, _CH)], idx_v)
        pltpu.async_copy(w_hbm.at[idx_v], rows_v, sem).wait()
        pltpu.sync_copy(rows_v, out_hbm.at[pl.ds(base, _CH)])
        return carry

    lax.fori_loop(0, _N_CH, chunk, 0)


@functools.cache
def _sc_gather():
    return functools.partial(
        pl.kernel,
        out_type=jax.ShapeDtypeStruct((_N_ROWS, _CODE_DIM), jnp.float32),
        mesh=plsc.VectorSubcoreMesh(core_axis_name="c", subcore_axis_name="s"),
        scratch_types=[
            pltpu.VMEM((_CH,), jnp.int32),
            pltpu.VMEM((_CH, _CODE_DIM), jnp.float32),
            pltpu.SemaphoreType.DMA,
        ],
        compiler_params=pltpu.CompilerParams(use_tc_tiling_on_sc=False),
    )(_sc_gather_body)


def kernel(inputs, W, usage_counts):
    flat3 = inputs.reshape(_N_BLOCKS, _BLK, _CODE_DIM)
    idx3, tot, com, ent, var, dec, ue = _tc_search(
        flat3, W, usage_counts.reshape(1, _NUM_CODES))
    idx_flat = idx3.reshape(_N_ROWS)
    quantized = _sc_gather()(W, idx_flat).reshape(inputs.shape)
    indices = idx3.reshape(inputs.shape[:-1])
    return (quantized, indices, tot.reshape(()), com.reshape(()),
            ent.reshape(()), var.reshape(()), dec.reshape(()),
            ue.reshape(()))


# speedup vs baseline: 1.6708x; 1.0057x over previous
import functools

import jax
import jax.numpy as jnp
import numpy as np
from jax import lax
from jax.experimental import pallas as pl
from jax.experimental.pallas import tpu as pltpu
from jax.experimental.pallas import tpu_sc as plsc

_NUM_CODES = 128
_CODE_DIM = 48
_COMMIT_W = 0.25
_EPS = 1e-05
_ENT_W = 0.1
_ENT_LO = 0.5
_ENT_HI = 0.9
_VAR_FLOOR = 0.05
_VAR_W = 0.001
_DECOR_W = 0.001

_N_ROWS = 128 * 1024
_BLK = 2048
_N_BLOCKS = _N_ROWS // _BLK


def _tc_body(x_ref, w_ref, u_ref, idx_ref, tot_ref, com_ref, ent_ref,
             var_ref, dec_ref, ue_ref, acc_ref):
    i = pl.program_id(0)
    x = x_ref[0]
    w = w_ref[...]

    xsq = jnp.sum(x * x, axis=1, keepdims=True)
    wsq = jnp.sum(w * w, axis=1)
    mm = jax.lax.dot_general(x, w, (((1,), (1,)), ((), ())),
                             preferred_element_type=jnp.float32)
    d = xsq - 2.0 * mm + wsq[None, :]

    mind = jnp.min(d, axis=1, keepdims=True)
    code_iota = lax.broadcasted_iota(jnp.int32, d.shape, 1)
    idx = jnp.min(jnp.where(d == mind, code_iota, _NUM_CODES), axis=1)
    idx_ref[0, 0, :] = idx

    blk_sum = jnp.sum(mind)

    @pl.when(i == 0)
    def _init():
        acc_ref[0] = blk_sum

    @pl.when(i > 0)
    def _acc():
        acc_ref[0] = acc_ref[0] + blk_sum

    @pl.when(i == _N_BLOCKS - 1)
    def _finalize():
        commit = _COMMIT_W * acc_ref[0] / float(_N_ROWS * _CODE_DIM)

        u = u_ref[...]
        p = u + _EPS
        p = p / jnp.maximum(jnp.sum(p), _EPS * _NUM_CODES)
        entropy = -jnp.sum(p * jnp.log(p + _EPS))
        ue = entropy / np.log(float(_NUM_CODES))
        gap = jnp.where(ue < _ENT_LO, _ENT_LO - ue,
                        jnp.where(ue > _ENT_HI, ue - _ENT_HI, 0.0))
        ent_loss = _ENT_W * gap * gap

        mean_w = jnp.mean(w, axis=0, keepdims=True)
        wc = w - mean_w
        variance = jnp.mean(wc * wc, axis=0, keepdims=True)
        var_loss = _VAR_W * jnp.mean(jnp.maximum(_VAR_FLOOR - variance, 0.0))

        cov = jax.lax.dot_general(wc, wc, (((0,), (0,)), ((), ())),
                                  preferred_element_type=jnp.float32)
        cov = cov / float(_NUM_CODES)
        ii = lax.broadcasted_iota(jnp.int32, cov.shape, 0)
        jj = lax.broadcasted_iota(jnp.int32, cov.shape, 1)
        off = jnp.where(ii == jj, 0.0, cov)
        dec_loss = _DECOR_W * jnp.sum(off * off) / float(_CODE_DIM * _CODE_DIM)

        tot_ref[...] = jnp.reshape(commit + ent_loss + var_loss + dec_loss,
                                   (1, 1))
        com_ref[...] = jnp.reshape(commit, (1, 1))
        ent_ref[...] = jnp.reshape(ent_loss, (1, 1))
        var_ref[...] = jnp.reshape(var_loss, (1, 1))
        dec_ref[...] = jnp.reshape(dec_loss, (1, 1))
        ue_ref[...] = jnp.reshape(ue, (1, 1))


def _tc_search(flat3, w, u2, interpret=False):
    scal = jax.ShapeDtypeStruct((1, 1), jnp.float32)
    return pl.pallas_call(
        _tc_body,
        grid=(_N_BLOCKS,),
        in_specs=[
            pl.BlockSpec((1, _BLK, _CODE_DIM), lambda i: (i, 0, 0)),
            pl.BlockSpec((_NUM_CODES, _CODE_DIM), lambda i: (0, 0)),
            pl.BlockSpec((1, _NUM_CODES), lambda i: (0, 0)),
        ],
        out_specs=[
            pl.BlockSpec((1, 1, _BLK), lambda i: (i, 0, 0)),
            pl.BlockSpec((1, 1), lambda i: (0, 0)),
            pl.BlockSpec((1, 1), lambda i: (0, 0)),
            pl.BlockSpec((1, 1), lambda i: (0, 0)),
            pl.BlockSpec((1, 1), lambda i: (0, 0)),
            pl.BlockSpec((1, 1), lambda i: (0, 0)),
            pl.BlockSpec((1, 1), lambda i: (0, 0)),
        ],
        out_shape=[
            jax.ShapeDtypeStruct((_N_BLOCKS, 1, _BLK), jnp.int32),
            scal, scal, scal, scal, scal, scal,
        ],
        scratch_shapes=[pltpu.SMEM((1,), jnp.float32)],
        interpret=interpret,
    )(flat3, w, u2)


_NW = 32
_B_PER_W = _N_ROWS // _NW
_CH = 512
_N_CH = _B_PER_W // _CH


_NB = 4


def _sc_gather_body(w_hbm, idx_hbm, out_hbm, idx_v, rows_v, *sems):
    gsems, wsems = sems[:_NB], sems[_NB:]
    wid = lax.axis_index("s") * 2 + lax.axis_index("c")
    base0 = wid * _B_PER_W
    pltpu.sync_copy(idx_hbm.at[pl.ds(base0, _B_PER_W)], idx_v)

    gops = [None] * _N_CH
    wops = [None] * _N_CH
    for c in range(_N_CH):
        b = c % _NB
        if c >= _NB:
            wops[c - _NB].wait()
        gops[c] = pltpu.async_copy(
            w_hbm.at[idx_v.at[pl.ds(c * _CH, _CH)]], rows_v.at[b], gsems[b])
        if c >= 1:
            gops[c - 1].wait()
            pb = (c - 1) % _NB
            wops[c - 1] = pltpu.async_copy(
                rows_v.at[pb], out_hbm.at[pl.ds(base0 + (c - 1) * _CH, _CH)],
                wsems[pb])
    gops[_N_CH - 1].wait()
    lb = (_N_CH - 1) % _NB
    wops[_N_CH - 1] = pltpu.async_copy(
        rows_v.at[lb], out_hbm.at[pl.ds(base0 + (_N_CH - 1) * _CH, _CH)],
        wsems[lb])
    for c in range(max(0, _N_CH - _NB), _N_CH):
        wops[c].wait()


@functools.cache
def _sc_gather():
    return functools.partial(
        pl.kernel,
        out_type=jax.ShapeDtypeStruct((_N_ROWS, _CODE_DIM), jnp.float32),
        mesh=plsc.VectorSubcoreMesh(core_axis_name="c", subcore_axis_name="s"),
        scratch_types=[
            pltpu.VMEM((_B_PER_W,), jnp.int32),
            pltpu.VMEM((_NB, _CH, _CODE_DIM), jnp.float32),
        ] + [pltpu.SemaphoreType.DMA] * (2 * _NB),
        compiler_params=pltpu.CompilerParams(use_tc_tiling_on_sc=False),
    )(_sc_gather_body)


def kernel(inputs, W, usage_counts):
    flat3 = inputs.reshape(_N_BLOCKS, _BLK, _CODE_DIM)
    idx3, tot, com, ent, var, dec, ue = _tc_search(
        flat3, W, usage_counts.reshape(1, _NUM_CODES))
    idx_flat = idx3.reshape(_N_ROWS)
    quantized = _sc_gather()(W, idx_flat).reshape(inputs.shape)
    indices = idx3.reshape(inputs.shape[:-1])
    return (quantized, indices, tot.reshape(()), com.reshape(()),
            ent.reshape(()), var.reshape(()), dec.reshape(()),
            ue.reshape(()))


# speedup vs baseline: 1.6751x; 1.0026x over previous
import functools

import jax
import jax.numpy as jnp
import numpy as np
from jax import lax
from jax.experimental import pallas as pl
from jax.experimental.pallas import tpu as pltpu
from jax.experimental.pallas import tpu_sc as plsc

_NUM_CODES = 128
_CODE_DIM = 48
_COMMIT_W = 0.25
_EPS = 1e-05
_ENT_W = 0.1
_ENT_LO = 0.5
_ENT_HI = 0.9
_VAR_FLOOR = 0.05
_VAR_W = 0.001
_DECOR_W = 0.001

_N_ROWS = 128 * 1024
_BLK = 2048
_N_BLOCKS = _N_ROWS // _BLK


def _tc_body(x_ref, w_ref, u_ref, idx_ref, tot_ref, com_ref, ent_ref,
             var_ref, dec_ref, ue_ref, acc_ref):
    i = pl.program_id(0)
    x = x_ref[0]
    w = w_ref[...]

    xsq = jnp.sum(x * x, axis=1, keepdims=True)
    wsq = jnp.sum(w * w, axis=1)
    mm = jax.lax.dot_general(x, w, (((1,), (1,)), ((), ())),
                             preferred_element_type=jnp.float32)
    d = xsq - 2.0 * mm + wsq[None, :]

    mind = jnp.min(d, axis=1, keepdims=True)
    code_iota = lax.broadcasted_iota(jnp.int32, d.shape, 1)
    idx = jnp.min(jnp.where(d == mind, code_iota, _NUM_CODES), axis=1)
    idx_ref[...] = idx

    blk_sum = jnp.sum(mind)

    @pl.when(i == 0)
    def _init():
        acc_ref[0] = blk_sum

    @pl.when(i > 0)
    def _acc():
        acc_ref[0] = acc_ref[0] + blk_sum

    @pl.when(i == _N_BLOCKS - 1)
    def _finalize():
        commit = _COMMIT_W * acc_ref[0] / float(_N_ROWS * _CODE_DIM)

        u = u_ref[...]
        p = u + _EPS
        p = p / jnp.maximum(jnp.sum(p), _EPS * _NUM_CODES)
        entropy = -jnp.sum(p * jnp.log(p + _EPS))
        ue = entropy / np.log(float(_NUM_CODES))
        gap = jnp.where(ue < _ENT_LO, _ENT_LO - ue,
                        jnp.where(ue > _ENT_HI, ue - _ENT_HI, 0.0))
        ent_loss = _ENT_W * gap * gap

        mean_w = jnp.mean(w, axis=0, keepdims=True)
        wc = w - mean_w
        variance = jnp.mean(wc * wc, axis=0, keepdims=True)
        var_loss = _VAR_W * jnp.mean(jnp.maximum(_VAR_FLOOR - variance, 0.0))

        cov = jax.lax.dot_general(wc, wc, (((0,), (0,)), ((), ())),
                                  preferred_element_type=jnp.float32)
        cov = cov / float(_NUM_CODES)
        ii = lax.broadcasted_iota(jnp.int32, cov.shape, 0)
        jj = lax.broadcasted_iota(jnp.int32, cov.shape, 1)
        off = jnp.where(ii == jj, 0.0, cov)
        dec_loss = _DECOR_W * jnp.sum(off * off) / float(_CODE_DIM * _CODE_DIM)

        tot_ref[...] = jnp.reshape(commit + ent_loss + var_loss + dec_loss,
                                   (1, 1))
        com_ref[...] = jnp.reshape(commit, (1, 1))
        ent_ref[...] = jnp.reshape(ent_loss, (1, 1))
        var_ref[...] = jnp.reshape(var_loss, (1, 1))
        dec_ref[...] = jnp.reshape(dec_loss, (1, 1))
        ue_ref[...] = jnp.reshape(ue, (1, 1))


def _tc_search(flat3, w, u2, interpret=False):
    scal = jax.ShapeDtypeStruct((1, 1), jnp.float32)
    return pl.pallas_call(
        _tc_body,
        grid=(_N_BLOCKS,),
        in_specs=[
            pl.BlockSpec((1, _BLK, _CODE_DIM), lambda i: (i, 0, 0)),
            pl.BlockSpec((_NUM_CODES, _CODE_DIM), lambda i: (0, 0)),
            pl.BlockSpec((1, _NUM_CODES), lambda i: (0, 0)),
        ],
        out_specs=[
            pl.BlockSpec((_BLK,), lambda i: (i,)),
            pl.BlockSpec((1, 1), lambda i: (0, 0)),
            pl.BlockSpec((1, 1), lambda i: (0, 0)),
            pl.BlockSpec((1, 1), lambda i: (0, 0)),
            pl.BlockSpec((1, 1), lambda i: (0, 0)),
            pl.BlockSpec((1, 1), lambda i: (0, 0)),
            pl.BlockSpec((1, 1), lambda i: (0, 0)),
        ],
        out_shape=[
            jax.ShapeDtypeStruct((_N_ROWS,), jnp.int32),
            scal, scal, scal, scal, scal, scal,
        ],
        scratch_shapes=[pltpu.SMEM((1,), jnp.float32)],
        interpret=interpret,
    )(flat3, w, u2)


_NW = 32
_B_PER_W = _N_ROWS // _NW
_CH = 512
_N_CH = _B_PER_W // _CH


_NB = 4


def _sc_gather_body(w_hbm, idx_hbm, out_hbm, idx_v, rows_v, *sems):
    gsems, wsems = sems[:_NB], sems[_NB:]
    wid = lax.axis_index("s") * 2 + lax.axis_index("c")
    base0 = wid * _B_PER_W
    pltpu.sync_copy(idx_hbm.at[pl.ds(base0, _B_PER_W)], idx_v)

    def _out_slice(c):
        g = base0 + c * _CH
        return out_hbm.at[g // 1024, pl.ds(g % 1024, _CH)]

    gops = [None] * _N_CH
    wops = [None] * _N_CH
    for c in range(_N_CH):
        b = c % _NB
        if c >= _NB:
            wops[c - _NB].wait()
        gops[c] = pltpu.async_copy(
            w_hbm.at[idx_v.at[pl.ds(c * _CH, _CH)]], rows_v.at[b], gsems[b])
        if c >= 1:
            gops[c - 1].wait()
            pb = (c - 1) % _NB
            wops[c - 1] = pltpu.async_copy(
                rows_v.at[pb], _out_slice(c - 1), wsems[pb])
    gops[_N_CH - 1].wait()
    lb = (_N_CH - 1) % _NB
    wops[_N_CH - 1] = pltpu.async_copy(
        rows_v.at[lb], _out_slice(_N_CH - 1), wsems[lb])
    for c in range(max(0, _N_CH - _NB), _N_CH):
        wops[c].wait()


@functools.cache
def _sc_gather():
    return functools.partial(
        pl.kernel,
        out_type=jax.ShapeDtypeStruct((128, 1024, _CODE_DIM), jnp.float32),
        mesh=plsc.VectorSubcoreMesh(core_axis_name="c", subcore_axis_name="s"),
        scratch_types=[
            pltpu.VMEM((_B_PER_W,), jnp.int32),
            pltpu.VMEM((_NB, _CH, _CODE_DIM), jnp.float32),
        ] + [pltpu.SemaphoreType.DMA] * (2 * _NB),
        compiler_params=pltpu.CompilerParams(use_tc_tiling_on_sc=False),
    )(_sc_gather_body)


def kernel(inputs, W, usage_counts):
    flat3 = inputs.reshape(_N_BLOCKS, _BLK, _CODE_DIM)
    idx_flat, tot, com, ent, var, dec, ue = _tc_search(
        flat3, W, usage_counts.reshape(1, _NUM_CODES))
    quantized = _sc_gather()(W, idx_flat)
    indices = idx_flat.reshape(inputs.shape[:-1])
    return (quantized, indices, tot.reshape(()), com.reshape(()),
            ent.reshape(()), var.reshape(()), dec.reshape(()),
            ue.reshape(()))


# speedup vs baseline: 1.7195x; 1.0265x over previous
import functools

import jax
import jax.numpy as jnp
import numpy as np
from jax import lax
from jax.experimental import pallas as pl
from jax.experimental.pallas import tpu as pltpu
from jax.experimental.pallas import tpu_sc as plsc

_NUM_CODES = 128
_CODE_DIM = 48
_COMMIT_W = 0.25
_EPS = 1e-05
_ENT_W = 0.1
_ENT_LO = 0.5
_ENT_HI = 0.9
_VAR_FLOOR = 0.05
_VAR_W = 0.001
_DECOR_W = 0.001

_N_ROWS = 128 * 1024
_BLK = 2048
_N_BLOCKS = _N_ROWS // _BLK


def _tc_body(x_ref, w_ref, u_ref, idx_ref, tot_ref, com_ref, ent_ref,
             var_ref, dec_ref, ue_ref, acc_ref):
    i = pl.program_id(0)
    x = x_ref[...].reshape(_BLK, _CODE_DIM)
    w = w_ref[...]

    xsq = jnp.sum(x * x, axis=1, keepdims=True)
    wsq = jnp.sum(w * w, axis=1)
    mm = jax.lax.dot_general(x, w, (((1,), (1,)), ((), ())),
                             preferred_element_type=jnp.float32)
    d = xsq - 2.0 * mm + wsq[None, :]

    dt = d.T
    mind = jnp.min(dt, axis=0, keepdims=True)
    code_iota = lax.broadcasted_iota(jnp.int32, dt.shape, 0)
    idx = jnp.min(jnp.where(dt == mind, code_iota, _NUM_CODES), axis=0)
    idx_ref[...] = idx

    blk_sum = jnp.sum(mind)

    @pl.when(i == 0)
    def _init():
        acc_ref[0] = blk_sum

    @pl.when(i > 0)
    def _acc():
        acc_ref[0] = acc_ref[0] + blk_sum

    @pl.when(i == _N_BLOCKS - 1)
    def _finalize():
        commit = _COMMIT_W * acc_ref[0] / float(_N_ROWS * _CODE_DIM)

        u = u_ref[...]
        p = u + _EPS
        p = p / jnp.maximum(jnp.sum(p), _EPS * _NUM_CODES)
        entropy = -jnp.sum(p * jnp.log(p + _EPS))
        ue = entropy / np.log(float(_NUM_CODES))
        gap = jnp.where(ue < _ENT_LO, _ENT_LO - ue,
                        jnp.where(ue > _ENT_HI, ue - _ENT_HI, 0.0))
        ent_loss = _ENT_W * gap * gap

        mean_w = jnp.mean(w, axis=0, keepdims=True)
        wc = w - mean_w
        variance = jnp.mean(wc * wc, axis=0, keepdims=True)
        var_loss = _VAR_W * jnp.mean(jnp.maximum(_VAR_FLOOR - variance, 0.0))

        cov = jax.lax.dot_general(wc, wc, (((0,), (0,)), ((), ())),
                                  preferred_element_type=jnp.float32)
        cov = cov / float(_NUM_CODES)
        ii = lax.broadcasted_iota(jnp.int32, cov.shape, 0)
        jj = lax.broadcasted_iota(jnp.int32, cov.shape, 1)
        off = jnp.where(ii == jj, 0.0, cov)
        dec_loss = _DECOR_W * jnp.sum(off * off) / float(_CODE_DIM * _CODE_DIM)

        tot_ref[...] = jnp.reshape(commit + ent_loss + var_loss + dec_loss,
                                   (1, 1))
        com_ref[...] = jnp.reshape(commit, (1, 1))
        ent_ref[...] = jnp.reshape(ent_loss, (1, 1))
        var_ref[...] = jnp.reshape(var_loss, (1, 1))
        dec_ref[...] = jnp.reshape(dec_loss, (1, 1))
        ue_ref[...] = jnp.reshape(ue, (1, 1))


def _tc_search(flat3, w, u2, interpret=False):
    scal = jax.ShapeDtypeStruct((1, 1), jnp.float32)
    return pl.pallas_call(
        _tc_body,
        grid=(_N_BLOCKS,),
        in_specs=[
            pl.BlockSpec((_BLK // 1024, 1024, _CODE_DIM), lambda i: (i, 0, 0)),
            pl.BlockSpec((_NUM_CODES, _CODE_DIM), lambda i: (0, 0)),
            pl.BlockSpec((1, _NUM_CODES), lambda i: (0, 0)),
        ],
        out_specs=[
            pl.BlockSpec((_BLK,), lambda i: (i,)),
            pl.BlockSpec((1, 1), lambda i: (0, 0)),
            pl.BlockSpec((1, 1), lambda i: (0, 0)),
            pl.BlockSpec((1, 1), lambda i: (0, 0)),
            pl.BlockSpec((1, 1), lambda i: (0, 0)),
            pl.BlockSpec((1, 1), lambda i: (0, 0)),
            pl.BlockSpec((1, 1), lambda i: (0, 0)),
        ],
        out_shape=[
            jax.ShapeDtypeStruct((_N_ROWS,), jnp.int32),
            scal, scal, scal, scal, scal, scal,
        ],
        scratch_shapes=[pltpu.SMEM((1,), jnp.float32)],
        interpret=interpret,
    )(flat3, w, u2)


_NW = 32
_B_PER_W = _N_ROWS // _NW
_CH = 512
_N_CH = _B_PER_W // _CH


_NB = 4


def _sc_gather_body(w_hbm, idx_hbm, out_hbm, idx_v, rows_v, *sems):
    gsems, wsems = sems[:_NB], sems[_NB:]
    wid = lax.axis_index("s") * 2 + lax.axis_index("c")
    base0 = wid * _B_PER_W
    pltpu.sync_copy(idx_hbm.at[pl.ds(base0, _B_PER_W)], idx_v)

    def _out_slice(c):
        g = base0 + c * _CH
        return out_hbm.at[g // 1024, pl.ds(g % 1024, _CH)]

    gops = [None] * _N_CH
    wops = [None] * _N_CH
    for c in range(_N_CH):
        b = c % _NB
        if c >= _NB:
            wops[c - _NB].wait()
        gops[c] = pltpu.async_copy(
            w_hbm.at[idx_v.at[pl.ds(c * _CH, _CH)]], rows_v.at[b], gsems[b])
        if c >= 1:
            gops[c - 1].wait()
            pb = (c - 1) % _NB
            wops[c - 1] = pltpu.async_copy(
                rows_v.at[pb], _out_slice(c - 1), wsems[pb])
    gops[_N_CH - 1].wait()
    lb = (_N_CH - 1) % _NB
    wops[_N_CH - 1] = pltpu.async_copy(
        rows_v.at[lb], _out_slice(_N_CH - 1), wsems[lb])
    for c in range(max(0, _N_CH - _NB), _N_CH):
        wops[c].wait()


@functools.cache
def _sc_gather():
    return functools.partial(
        pl.kernel,
        out_type=jax.ShapeDtypeStruct((128, 1024, _CODE_DIM), jnp.float32),
        mesh=plsc.VectorSubcoreMesh(core_axis_name="c", subcore_axis_name="s"),
        scratch_types=[
            pltpu.VMEM((_B_PER_W,), jnp.int32),
            pltpu.VMEM((_NB, _CH, _CODE_DIM), jnp.float32),
        ] + [pltpu.SemaphoreType.DMA] * (2 * _NB),
        compiler_params=pltpu.CompilerParams(use_tc_tiling_on_sc=False),
    )(_sc_gather_body)


def kernel(inputs, W, usage_counts):
    idx_flat, tot, com, ent, var, dec, ue = _tc_search(
        inputs, W, usage_counts.reshape(1, _NUM_CODES))
    quantized = _sc_gather()(W, idx_flat)
    indices = idx_flat.reshape(inputs.shape[:-1])
    return (quantized, indices, tot.reshape(()), com.reshape(()),
            ent.reshape(()), var.reshape(()), dec.reshape(()),
            ue.reshape(()))


# speedup vs baseline: 2.0002x; 1.1632x over previous
import functools

import jax
import jax.numpy as jnp
import numpy as np
from jax import lax
from jax.experimental import pallas as pl
from jax.experimental.pallas import tpu as pltpu
from jax.experimental.pallas import tpu_sc as plsc

_NUM_CODES = 128
_CODE_DIM = 48
_COMMIT_W = 0.25
_EPS = 1e-05
_ENT_W = 0.1
_ENT_LO = 0.5
_ENT_HI = 0.9
_VAR_FLOOR = 0.05
_VAR_W = 0.001
_DECOR_W = 0.001

_N_ROWS = 128 * 1024
_BLK = 2048
_N_BLOCKS = _N_ROWS // _BLK


def _tc_body(x_ref, w_ref, u_ref, idx_ref, tot_ref, com_ref, ent_ref,
             var_ref, dec_ref, ue_ref, acc_ref):
    i = pl.program_id(0)
    x = x_ref[...].reshape(_BLK, _CODE_DIM)
    w = w_ref[...]

    xsq = jnp.sum(x * x, axis=1, keepdims=True)
    wsq = jnp.sum(w * w, axis=1)
    mm = jax.lax.dot_general(x, w, (((1,), (1,)), ((), ())),
                             preferred_element_type=jnp.float32)
    d = xsq - 2.0 * mm + wsq[None, :]

    dt = d.T
    mind = jnp.min(dt, axis=0, keepdims=True)
    code_iota = lax.broadcasted_iota(jnp.int32, dt.shape, 0)
    idx = jnp.min(jnp.where(dt == mind, code_iota, _NUM_CODES), axis=0)
    idx_ref[...] = idx

    blk_sum = jnp.sum(mind)

    @pl.when(i == 0)
    def _init():
        acc_ref[0] = blk_sum

    @pl.when(i > 0)
    def _acc():
        acc_ref[0] = acc_ref[0] + blk_sum

    @pl.when(i == _N_BLOCKS - 1)
    def _finalize():
        commit = _COMMIT_W * acc_ref[0] / float(_N_ROWS * _CODE_DIM)

        u = u_ref[...]
        p = u + _EPS
        p = p / jnp.maximum(jnp.sum(p), _EPS * _NUM_CODES)
        entropy = -jnp.sum(p * jnp.log(p + _EPS))
        ue = entropy / np.log(float(_NUM_CODES))
        gap = jnp.where(ue < _ENT_LO, _ENT_LO - ue,
                        jnp.where(ue > _ENT_HI, ue - _ENT_HI, 0.0))
        ent_loss = _ENT_W * gap * gap

        mean_w = jnp.mean(w, axis=0, keepdims=True)
        wc = w - mean_w
        variance = jnp.mean(wc * wc, axis=0, keepdims=True)
        var_loss = _VAR_W * jnp.mean(jnp.maximum(_VAR_FLOOR - variance, 0.0))

        cov = jax.lax.dot_general(wc, wc, (((0,), (0,)), ((), ())),
                                  preferred_element_type=jnp.float32)
        cov = cov / float(_NUM_CODES)
        ii = lax.broadcasted_iota(jnp.int32, cov.shape, 0)
        jj = lax.broadcasted_iota(jnp.int32, cov.shape, 1)
        off = jnp.where(ii == jj, 0.0, cov)
        dec_loss = _DECOR_W * jnp.sum(off * off) / float(_CODE_DIM * _CODE_DIM)

        tot_ref[...] = jnp.reshape(commit + ent_loss + var_loss + dec_loss,
                                   (1, 1))
        com_ref[...] = jnp.reshape(commit, (1, 1))
        ent_ref[...] = jnp.reshape(ent_loss, (1, 1))
        var_ref[...] = jnp.reshape(var_loss, (1, 1))
        dec_ref[...] = jnp.reshape(dec_loss, (1, 1))
        ue_ref[...] = jnp.reshape(ue, (1, 1))


def _tc_search(flat3, w, u2, interpret=False):
    scal = jax.ShapeDtypeStruct((1, 1), jnp.float32)
    return pl.pallas_call(
        _tc_body,
        grid=(_N_BLOCKS,),
        in_specs=[
            pl.BlockSpec((1, _BLK, _CODE_DIM), lambda i: (i, 0, 0)),
            pl.BlockSpec((_NUM_CODES, _CODE_DIM), lambda i: (0, 0)),
            pl.BlockSpec((1, _NUM_CODES), lambda i: (0, 0)),
        ],
        out_specs=[
            pl.BlockSpec((_BLK,), lambda i: (i,)),
            pl.BlockSpec((1, 1), lambda i: (0, 0)),
            pl.BlockSpec((1, 1), lambda i: (0, 0)),
            pl.BlockSpec((1, 1), lambda i: (0, 0)),
            pl.BlockSpec((1, 1), lambda i: (0, 0)),
            pl.BlockSpec((1, 1), lambda i: (0, 0)),
            pl.BlockSpec((1, 1), lambda i: (0, 0)),
        ],
        out_shape=[
            jax.ShapeDtypeStruct((_N_ROWS,), jnp.int32),
            scal, scal, scal, scal, scal, scal,
        ],
        scratch_shapes=[pltpu.SMEM((1,), jnp.float32)],
        compiler_params=pltpu.CompilerParams(
            allow_input_fusion=[True, False, False]),
        interpret=interpret,
    )(flat3, w, u2)


_NW = 32
_B_PER_W = _N_ROWS // _NW
_CH = 512
_N_CH = _B_PER_W // _CH


_NB = 4


def _sc_gather_body(w_hbm, idx_hbm, out_hbm, idx_v, rows_v, *sems):
    gsems, wsems = sems[:_NB], sems[_NB:]
    wid = lax.axis_index("s") * 2 + lax.axis_index("c")
    base0 = wid * _B_PER_W
    pltpu.sync_copy(idx_hbm.at[pl.ds(base0, _B_PER_W)], idx_v)

    def _out_slice(c):
        g = base0 + c * _CH
        return out_hbm.at[g // 1024, pl.ds(g % 1024, _CH)]

    gops = [None] * _N_CH
    wops = [None] * _N_CH
    for c in range(_N_CH):
        b = c % _NB
        if c >= _NB:
            wops[c - _NB].wait()
        gops[c] = pltpu.async_copy(
            w_hbm.at[idx_v.at[pl.ds(c * _CH, _CH)]], rows_v.at[b], gsems[b])
        if c >= 1:
            gops[c - 1].wait()
            pb = (c - 1) % _NB
            wops[c - 1] = pltpu.async_copy(
                rows_v.at[pb], _out_slice(c - 1), wsems[pb])
    gops[_N_CH - 1].wait()
    lb = (_N_CH - 1) % _NB
    wops[_N_CH - 1] = pltpu.async_copy(
        rows_v.at[lb], _out_slice(_N_CH - 1), wsems[lb])
    for c in range(max(0, _N_CH - _NB), _N_CH):
        wops[c].wait()


@functools.cache
def _sc_gather():
    return functools.partial(
        pl.kernel,
        out_type=jax.ShapeDtypeStruct((128, 1024, _CODE_DIM), jnp.float32),
        mesh=plsc.VectorSubcoreMesh(core_axis_name="c", subcore_axis_name="s"),
        scratch_types=[
            pltpu.VMEM((_B_PER_W,), jnp.int32),
            pltpu.VMEM((_NB, _CH, _CODE_DIM), jnp.float32),
        ] + [pltpu.SemaphoreType.DMA] * (2 * _NB),
        compiler_params=pltpu.CompilerParams(use_tc_tiling_on_sc=False),
    )(_sc_gather_body)


def kernel(inputs, W, usage_counts):
    flat3 = inputs.reshape(_N_BLOCKS, _BLK, _CODE_DIM)
    idx_flat, tot, com, ent, var, dec, ue = _tc_search(
        flat3, W, usage_counts.reshape(1, _NUM_CODES))
    quantized = _sc_gather()(W, idx_flat)
    indices = idx_flat.reshape(inputs.shape[:-1])
    return (quantized, indices, tot.reshape(()), com.reshape(()),
            ent.reshape(()), var.reshape(()), dec.reshape(()),
            ue.reshape(()))


# speedup vs baseline: 3.4686x; 1.7342x over previous
import functools

import jax
import jax.numpy as jnp
import numpy as np
from jax import lax
from jax.experimental import pallas as pl
from jax.experimental.pallas import tpu as pltpu
from jax.experimental.pallas import tpu_sc as plsc

_NUM_CODES = 128
_CODE_DIM = 48
_COMMIT_W = 0.25
_EPS = 1e-05
_ENT_W = 0.1
_ENT_LO = 0.5
_ENT_HI = 0.9
_VAR_FLOOR = 0.05
_VAR_W = 0.001
_DECOR_W = 0.001

_N_ROWS = 128 * 1024
_BLK = 2048
_N_BLOCKS = _N_ROWS // _BLK


def _tc_body(x_ref, w_ref, u_ref, idx_ref, q_ref, tot_ref, com_ref, ent_ref,
             var_ref, dec_ref, ue_ref, acc_ref):
    i = pl.program_id(0)
    x = x_ref[...].reshape(_BLK, _CODE_DIM)
    w = w_ref[...]

    xsq = jnp.sum(x * x, axis=1, keepdims=True)
    wsq = jnp.sum(w * w, axis=1)
    mm = jax.lax.dot_general(x, w, (((1,), (1,)), ((), ())),
                             preferred_element_type=jnp.float32)
    d = xsq - 2.0 * mm + wsq[None, :]

    dt = d.T
    mind = jnp.min(dt, axis=0, keepdims=True)
    code_iota = lax.broadcasted_iota(jnp.int32, dt.shape, 0)
    idx = jnp.min(jnp.where(dt == mind, code_iota, _NUM_CODES), axis=0)
    idx_ref[...] = idx

    onehot_t = (code_iota == idx[None, :]).astype(jnp.float32)
    q = jax.lax.dot_general(onehot_t, w, (((0,), (0,)), ((), ())),
                            preferred_element_type=jnp.float32)
    q_ref[...] = q.reshape(q_ref.shape)

    blk_sum = jnp.sum(mind)

    @pl.when(i == 0)
    def _init():
        acc_ref[0] = blk_sum

    @pl.when(i > 0)
    def _acc():
        acc_ref[0] = acc_ref[0] + blk_sum

    @pl.when(i == _N_BLOCKS - 1)
    def _finalize():
        commit = _COMMIT_W * acc_ref[0] / float(_N_ROWS * _CODE_DIM)

        u = u_ref[...]
        p = u + _EPS
        p = p / jnp.maximum(jnp.sum(p), _EPS * _NUM_CODES)
        entropy = -jnp.sum(p * jnp.log(p + _EPS))
        ue = entropy / np.log(float(_NUM_CODES))
        gap = jnp.where(ue < _ENT_LO, _ENT_LO - ue,
                        jnp.where(ue > _ENT_HI, ue - _ENT_HI, 0.0))
        ent_loss = _ENT_W * gap * gap

        mean_w = jnp.mean(w, axis=0, keepdims=True)
        wc = w - mean_w
        variance = jnp.mean(wc * wc, axis=0, keepdims=True)
        var_loss = _VAR_W * jnp.mean(jnp.maximum(_VAR_FLOOR - variance, 0.0))

        cov = jax.lax.dot_general(wc, wc, (((0,), (0,)), ((), ())),
                                  preferred_element_type=jnp.float32)
        cov = cov / float(_NUM_CODES)
        ii = lax.broadcasted_iota(jnp.int32, cov.shape, 0)
        jj = lax.broadcasted_iota(jnp.int32, cov.shape, 1)
        off = jnp.where(ii == jj, 0.0, cov)
        dec_loss = _DECOR_W * jnp.sum(off * off) / float(_CODE_DIM * _CODE_DIM)

        tot_ref[...] = jnp.reshape(commit + ent_loss + var_loss + dec_loss,
                                   (1, 1))
        com_ref[...] = jnp.reshape(commit, (1, 1))
        ent_ref[...] = jnp.reshape(ent_loss, (1, 1))
        var_ref[...] = jnp.reshape(var_loss, (1, 1))
        dec_ref[...] = jnp.reshape(dec_loss, (1, 1))
        ue_ref[...] = jnp.reshape(ue, (1, 1))


def _tc_search(flat3, w, u2, interpret=False):
    scal = jax.ShapeDtypeStruct((1, 1), jnp.float32)
    return pl.pallas_call(
        _tc_body,
        grid=(_N_BLOCKS,),
        in_specs=[
            pl.BlockSpec((1, _BLK, _CODE_DIM), lambda i: (i, 0, 0)),
            pl.BlockSpec((_NUM_CODES, _CODE_DIM), lambda i: (0, 0)),
            pl.BlockSpec((1, _NUM_CODES), lambda i: (0, 0)),
        ],
        out_specs=[
            pl.BlockSpec((_BLK,), lambda i: (i,)),
            pl.BlockSpec((1, _BLK, _CODE_DIM), lambda i: (i, 0, 0)),
            pl.BlockSpec((1, 1), lambda i: (0, 0)),
            pl.BlockSpec((1, 1), lambda i: (0, 0)),
            pl.BlockSpec((1, 1), lambda i: (0, 0)),
            pl.BlockSpec((1, 1), lambda i: (0, 0)),
            pl.BlockSpec((1, 1), lambda i: (0, 0)),
            pl.BlockSpec((1, 1), lambda i: (0, 0)),
        ],
        out_shape=[
            jax.ShapeDtypeStruct((_N_ROWS,), jnp.int32),
            jax.ShapeDtypeStruct((_N_BLOCKS, _BLK, _CODE_DIM), jnp.float32),
            scal, scal, scal, scal, scal, scal,
        ],
        scratch_shapes=[pltpu.SMEM((1,), jnp.float32)],
        compiler_params=pltpu.CompilerParams(
            allow_input_fusion=[True, False, False]),
        interpret=interpret,
    )(flat3, w, u2)


_NW = 32
_B_PER_W = _N_ROWS // _NW
_CH = 512
_N_CH = _B_PER_W // _CH


_NB = 4


def _sc_gather_body(w_hbm, idx_hbm, out_hbm, idx_v, rows_v, *sems):
    gsems, wsems = sems[:_NB], sems[_NB:]
    wid = lax.axis_index("s") * 2 + lax.axis_index("c")
    base0 = wid * _B_PER_W
    pltpu.sync_copy(idx_hbm.at[pl.ds(base0, _B_PER_W)], idx_v)

    def _out_slice(c):
        g = base0 + c * _CH
        return out_hbm.at[g // 1024, pl.ds(g % 1024, _CH)]

    gops = [None] * _N_CH
    wops = [None] * _N_CH
    for c in range(_N_CH):
        b = c % _NB
        if c >= _NB:
            wops[c - _NB].wait()
        gops[c] = pltpu.async_copy(
            w_hbm.at[idx_v.at[pl.ds(c * _CH, _CH)]], rows_v.at[b], gsems[b])
        if c >= 1:
            gops[c - 1].wait()
            pb = (c - 1) % _NB
            wops[c - 1] = pltpu.async_copy(
                rows_v.at[pb], _out_slice(c - 1), wsems[pb])
    gops[_N_CH - 1].wait()
    lb = (_N_CH - 1) % _NB
    wops[_N_CH - 1] = pltpu.async_copy(
        rows_v.at[lb], _out_slice(_N_CH - 1), wsems[lb])
    for c in range(max(0, _N_CH - _NB), _N_CH):
        wops[c].wait()


@functools.cache
def _sc_gather():
    return functools.partial(
        pl.kernel,
        out_type=jax.ShapeDtypeStruct((128, 1024, _CODE_DIM), jnp.float32),
        mesh=plsc.VectorSubcoreMesh(core_axis_name="c", subcore_axis_name="s"),
        scratch_types=[
            pltpu.VMEM((_B_PER_W,), jnp.int32),
            pltpu.VMEM((_NB, _CH, _CODE_DIM), jnp.float32),
        ] + [pltpu.SemaphoreType.DMA] * (2 * _NB),
        compiler_params=pltpu.CompilerParams(use_tc_tiling_on_sc=False),
    )(_sc_gather_body)


def kernel(inputs, W, usage_counts):
    flat3 = inputs.reshape(_N_BLOCKS, _BLK, _CODE_DIM)
    idx_flat, q3, tot, com, ent, var, dec, ue = _tc_search(
        flat3, W, usage_counts.reshape(1, _NUM_CODES))
    quantized = q3.reshape(inputs.shape)
    indices = idx_flat.reshape(inputs.shape[:-1])
    return (quantized, indices, tot.reshape(()), com.reshape(()),
            ent.reshape(()), var.reshape(()), dec.reshape(()),
            ue.reshape(()))


# speedup vs baseline: 3.9320x; 1.1336x over previous
import functools

import jax
import jax.numpy as jnp
import numpy as np
from jax import lax
from jax.experimental import pallas as pl
from jax.experimental.pallas import tpu as pltpu
from jax.experimental.pallas import tpu_sc as plsc

_NUM_CODES = 128
_CODE_DIM = 48
_COMMIT_W = 0.25
_EPS = 1e-05
_ENT_W = 0.1
_ENT_LO = 0.5
_ENT_HI = 0.9
_VAR_FLOOR = 0.05
_VAR_W = 0.001
_DECOR_W = 0.001

_N_ROWS = 128 * 1024
_BLK = 4096
_N_BLOCKS = _N_ROWS // _BLK


def _tc_body(x_ref, w_ref, u_ref, idx_ref, q_ref, tot_ref, com_ref, ent_ref,
             var_ref, dec_ref, ue_ref, acc_ref):
    i = pl.program_id(0)
    x = x_ref[...].reshape(_BLK, _CODE_DIM)
    w = w_ref[...]

    xsq = jnp.sum(x * x, axis=1, keepdims=True)
    wsq = jnp.sum(w * w, axis=1)
    mm = jax.lax.dot_general(x, w, (((1,), (1,)), ((), ())),
                             preferred_element_type=jnp.float32)
    d = xsq - 2.0 * mm + wsq[None, :]

    dt = d.T
    mind = jnp.min(dt, axis=0, keepdims=True)
    code_iota = lax.broadcasted_iota(jnp.int32, dt.shape, 0)
    idx = jnp.min(jnp.where(dt == mind, code_iota, _NUM_CODES), axis=0)
    idx_ref[...] = idx

    onehot_t = (code_iota == idx[None, :]).astype(jnp.float32)
    q = jax.lax.dot_general(onehot_t, w, (((0,), (0,)), ((), ())),
                            preferred_element_type=jnp.float32)
    q_ref[...] = q.reshape(q_ref.shape)

    blk_sum = jnp.sum(mind)

    @pl.when(i == 0)
    def _init():
        acc_ref[0] = blk_sum

    @pl.when(i > 0)
    def _acc():
        acc_ref[0] = acc_ref[0] + blk_sum

    @pl.when(i == _N_BLOCKS - 1)
    def _finalize():
        commit = _COMMIT_W * acc_ref[0] / float(_N_ROWS * _CODE_DIM)

        u = u_ref[...]
        p = u + _EPS
        p = p / jnp.maximum(jnp.sum(p), _EPS * _NUM_CODES)
        entropy = -jnp.sum(p * jnp.log(p + _EPS))
        ue = entropy / np.log(float(_NUM_CODES))
        gap = jnp.where(ue < _ENT_LO, _ENT_LO - ue,
                        jnp.where(ue > _ENT_HI, ue - _ENT_HI, 0.0))
        ent_loss = _ENT_W * gap * gap

        mean_w = jnp.mean(w, axis=0, keepdims=True)
        wc = w - mean_w
        variance = jnp.mean(wc * wc, axis=0, keepdims=True)
        var_loss = _VAR_W * jnp.mean(jnp.maximum(_VAR_FLOOR - variance, 0.0))

        cov = jax.lax.dot_general(wc, wc, (((0,), (0,)), ((), ())),
                                  preferred_element_type=jnp.float32)
        cov = cov / float(_NUM_CODES)
        ii = lax.broadcasted_iota(jnp.int32, cov.shape, 0)
        jj = lax.broadcasted_iota(jnp.int32, cov.shape, 1)
        off = jnp.where(ii == jj, 0.0, cov)
        dec_loss = _DECOR_W * jnp.sum(off * off) / float(_CODE_DIM * _CODE_DIM)

        tot_ref[...] = jnp.reshape(commit + ent_loss + var_loss + dec_loss,
                                   (1, 1))
        com_ref[...] = jnp.reshape(commit, (1, 1))
        ent_ref[...] = jnp.reshape(ent_loss, (1, 1))
        var_ref[...] = jnp.reshape(var_loss, (1, 1))
        dec_ref[...] = jnp.reshape(dec_loss, (1, 1))
        ue_ref[...] = jnp.reshape(ue, (1, 1))


def _tc_search(flat3, w, u2, interpret=False):
    scal = jax.ShapeDtypeStruct((1, 1), jnp.float32)
    return pl.pallas_call(
        _tc_body,
        grid=(_N_BLOCKS,),
        in_specs=[
            pl.BlockSpec((1, _BLK, _CODE_DIM), lambda i: (i, 0, 0)),
            pl.BlockSpec((_NUM_CODES, _CODE_DIM), lambda i: (0, 0)),
            pl.BlockSpec((1, _NUM_CODES), lambda i: (0, 0)),
        ],
        out_specs=[
            pl.BlockSpec((_BLK,), lambda i: (i,)),
            pl.BlockSpec((1, _BLK, _CODE_DIM), lambda i: (i, 0, 0)),
            pl.BlockSpec((1, 1), lambda i: (0, 0)),
            pl.BlockSpec((1, 1), lambda i: (0, 0)),
            pl.BlockSpec((1, 1), lambda i: (0, 0)),
            pl.BlockSpec((1, 1), lambda i: (0, 0)),
            pl.BlockSpec((1, 1), lambda i: (0, 0)),
            pl.BlockSpec((1, 1), lambda i: (0, 0)),
        ],
        out_shape=[
            jax.ShapeDtypeStruct((_N_ROWS,), jnp.int32),
            jax.ShapeDtypeStruct((_N_BLOCKS, _BLK, _CODE_DIM), jnp.float32),
            scal, scal, scal, scal, scal, scal,
        ],
        scratch_shapes=[pltpu.SMEM((1,), jnp.float32)],
        compiler_params=pltpu.CompilerParams(
            allow_input_fusion=[True, False, False]),
        interpret=interpret,
    )(flat3, w, u2)


_NW = 32
_B_PER_W = _N_ROWS // _NW
_CH = 512
_N_CH = _B_PER_W // _CH


_NB = 4


def _sc_gather_body(w_hbm, idx_hbm, out_hbm, idx_v, rows_v, *sems):
    gsems, wsems = sems[:_NB], sems[_NB:]
    wid = lax.axis_index("s") * 2 + lax.axis_index("c")
    base0 = wid * _B_PER_W
    pltpu.sync_copy(idx_hbm.at[pl.ds(base0, _B_PER_W)], idx_v)

    def _out_slice(c):
        g = base0 + c * _CH
        return out_hbm.at[g // 1024, pl.ds(g % 1024, _CH)]

    gops = [None] * _N_CH
    wops = [None] * _N_CH
    for c in range(_N_CH):
        b = c % _NB
        if c >= _NB:
            wops[c - _NB].wait()
        gops[c] = pltpu.async_copy(
            w_hbm.at[idx_v.at[pl.ds(c * _CH, _CH)]], rows_v.at[b], gsems[b])
        if c >= 1:
            gops[c - 1].wait()
            pb = (c - 1) % _NB
            wops[c - 1] = pltpu.async_copy(
                rows_v.at[pb], _out_slice(c - 1), wsems[pb])
    gops[_N_CH - 1].wait()
    lb = (_N_CH - 1) % _NB
    wops[_N_CH - 1] = pltpu.async_copy(
        rows_v.at[lb], _out_slice(_N_CH - 1), wsems[lb])
    for c in range(max(0, _N_CH - _NB), _N_CH):
        wops[c].wait()


@functools.cache
def _sc_gather():
    return functools.partial(
        pl.kernel,
        out_type=jax.ShapeDtypeStruct((128, 1024, _CODE_DIM), jnp.float32),
        mesh=plsc.VectorSubcoreMesh(core_axis_name="c", subcore_axis_name="s"),
        scratch_types=[
            pltpu.VMEM((_B_PER_W,), jnp.int32),
            pltpu.VMEM((_NB, _CH, _CODE_DIM), jnp.float32),
        ] + [pltpu.SemaphoreType.DMA] * (2 * _NB),
        compiler_params=pltpu.CompilerParams(use_tc_tiling_on_sc=False),
    )(_sc_gather_body)


_CH2 = 256
_NB2 = 4


def _sc_gather2_body(w_hbm, idx_hbm, out_hbm, idx_v, *sems):
    pl.run_scoped(
        functools.partial(_sc_gather2_inner, w_hbm, idx_hbm, out_hbm, idx_v,
                          sems),
        plsc.MemoryRef((_NB2, _CH2, 128), jnp.float32, pltpu.VMEM,
                       tiling=(8, 128)),
    )


def _sc_gather2_inner(w_hbm, idx_hbm, out_hbm, idx_v, sems, rows_v):
    gsems, wsems = sems[:_NB2], sems[_NB2:]
    wid = lax.axis_index("s") * 2 + lax.axis_index("c")
    base0 = wid * _B_PER_W
    pltpu.sync_copy(idx_hbm.at[pl.ds(base0, _B_PER_W)], idx_v)
    n_ch = _B_PER_W // _CH2

    def _out_slice(c):
        g = base0 + c * _CH2
        return out_hbm.at[g // 1024, pl.ds(g % 1024, _CH2)]

    gops = [None] * n_ch
    wops = [None] * n_ch
    for c in range(n_ch):
        b = c % _NB2
        if c >= _NB2:
            wops[c - _NB2].wait()
        gops[c] = pltpu.async_copy(
            w_hbm.at[idx_v.at[pl.ds(c * _CH2, _CH2)]], rows_v.at[b], gsems[b])
        if c >= 1:
            gops[c - 1].wait()
            pb = (c - 1) % _NB2
            wops[c - 1] = pltpu.async_copy(
                rows_v.at[pb, :, pl.ds(0, _CODE_DIM)], _out_slice(c - 1),
                wsems[pb])
    gops[n_ch - 1].wait()
    lb = (n_ch - 1) % _NB2
    wops[n_ch - 1] = pltpu.async_copy(
        rows_v.at[lb, :, pl.ds(0, _CODE_DIM)], _out_slice(n_ch - 1), wsems[lb])
    for c in range(max(0, n_ch - _NB2), n_ch):
        wops[c].wait()


@functools.cache
def _sc_gather2():
    return functools.partial(
        pl.kernel,
        out_type=jax.ShapeDtypeStruct((128, 1024, _CODE_DIM), jnp.float32),
        mesh=plsc.VectorSubcoreMesh(core_axis_name="c", subcore_axis_name="s"),
        scratch_types=[
            pltpu.VMEM((_B_PER_W,), jnp.int32),
        ] + [pltpu.SemaphoreType.DMA] * (2 * _NB2),
    )(_sc_gather2_body)


def kernel(inputs, W, usage_counts):
    flat3 = inputs.reshape(_N_BLOCKS, _BLK, _CODE_DIM)
    idx_flat, q3, tot, com, ent, var, dec, ue = _tc_search(
        flat3, W, usage_counts.reshape(1, _NUM_CODES))
    quantized = q3.reshape(inputs.shape)
    indices = idx_flat.reshape(inputs.shape[:-1])
    return (quantized, indices, tot.reshape(()), com.reshape(()),
            ent.reshape(()), var.reshape(()), dec.reshape(()),
            ue.reshape(()))


# speedup vs baseline: 4.1969x; 1.0674x over previous
import functools

import jax
import jax.numpy as jnp
import numpy as np
from jax import lax
from jax.experimental import pallas as pl
from jax.experimental.pallas import tpu as pltpu
from jax.experimental.pallas import tpu_sc as plsc

_NUM_CODES = 128
_CODE_DIM = 48
_COMMIT_W = 0.25
_EPS = 1e-05
_ENT_W = 0.1
_ENT_LO = 0.5
_ENT_HI = 0.9
_VAR_FLOOR = 0.05
_VAR_W = 0.001
_DECOR_W = 0.001

_N_ROWS = 128 * 1024
_BLK = 8192
_N_BLOCKS = _N_ROWS // _BLK


def _tc_body(x_ref, w_ref, u_ref, idx_ref, q_ref, tot_ref, com_ref, ent_ref,
             var_ref, dec_ref, ue_ref, acc_ref):
    i = pl.program_id(0)
    x = x_ref[...].reshape(_BLK, _CODE_DIM)
    w = w_ref[...]

    xsq = jnp.sum(x * x, axis=1, keepdims=True)
    wsq = jnp.sum(w * w, axis=1)
    mm = jax.lax.dot_general(x, w, (((1,), (1,)), ((), ())),
                             preferred_element_type=jnp.float32)
    d = xsq - 2.0 * mm + wsq[None, :]

    dt = d.T
    mind = jnp.min(dt, axis=0, keepdims=True)
    code_iota = lax.broadcasted_iota(jnp.int32, dt.shape, 0)
    idx = jnp.min(jnp.where(dt == mind, code_iota, _NUM_CODES), axis=0)
    idx_ref[...] = idx

    onehot_t = (code_iota == idx[None, :]).astype(jnp.float32)
    q = jax.lax.dot_general(onehot_t, w, (((0,), (0,)), ((), ())),
                            preferred_element_type=jnp.float32)
    q_ref[...] = q.reshape(q_ref.shape)

    blk_sum = jnp.sum(mind)

    @pl.when(i == 0)
    def _init():
        acc_ref[0] = blk_sum

    @pl.when(i > 0)
    def _acc():
        acc_ref[0] = acc_ref[0] + blk_sum

    @pl.when(i == _N_BLOCKS - 1)
    def _finalize():
        commit = _COMMIT_W * acc_ref[0] / float(_N_ROWS * _CODE_DIM)

        u = u_ref[...]
        p = u + _EPS
        p = p / jnp.maximum(jnp.sum(p), _EPS * _NUM_CODES)
        entropy = -jnp.sum(p * jnp.log(p + _EPS))
        ue = entropy / np.log(float(_NUM_CODES))
        gap = jnp.where(ue < _ENT_LO, _ENT_LO - ue,
                        jnp.where(ue > _ENT_HI, ue - _ENT_HI, 0.0))
        ent_loss = _ENT_W * gap * gap

        mean_w = jnp.mean(w, axis=0, keepdims=True)
        wc = w - mean_w
        variance = jnp.mean(wc * wc, axis=0, keepdims=True)
        var_loss = _VAR_W * jnp.mean(jnp.maximum(_VAR_FLOOR - variance, 0.0))

        cov = jax.lax.dot_general(wc, wc, (((0,), (0,)), ((), ())),
                                  preferred_element_type=jnp.float32)
        cov = cov / float(_NUM_CODES)
        ii = lax.broadcasted_iota(jnp.int32, cov.shape, 0)
        jj = lax.broadcasted_iota(jnp.int32, cov.shape, 1)
        off = jnp.where(ii == jj, 0.0, cov)
        dec_loss = _DECOR_W * jnp.sum(off * off) / float(_CODE_DIM * _CODE_DIM)

        tot_ref[...] = jnp.reshape(commit + ent_loss + var_loss + dec_loss,
                                   (1, 1))
        com_ref[...] = jnp.reshape(commit, (1, 1))
        ent_ref[...] = jnp.reshape(ent_loss, (1, 1))
        var_ref[...] = jnp.reshape(var_loss, (1, 1))
        dec_ref[...] = jnp.reshape(dec_loss, (1, 1))
        ue_ref[...] = jnp.reshape(ue, (1, 1))


def _tc_search(flat3, w, u2, interpret=False):
    scal = jax.ShapeDtypeStruct((1, 1), jnp.float32)
    return pl.pallas_call(
        _tc_body,
        grid=(_N_BLOCKS,),
        in_specs=[
            pl.BlockSpec((1, _BLK, _CODE_DIM), lambda i: (i, 0, 0)),
            pl.BlockSpec((_NUM_CODES, _CODE_DIM), lambda i: (0, 0)),
            pl.BlockSpec((1, _NUM_CODES), lambda i: (0, 0)),
        ],
        out_specs=[
            pl.BlockSpec((_BLK,), lambda i: (i,)),
            pl.BlockSpec((1, _BLK, _CODE_DIM), lambda i: (i, 0, 0)),
            pl.BlockSpec((1, 1), lambda i: (0, 0)),
            pl.BlockSpec((1, 1), lambda i: (0, 0)),
            pl.BlockSpec((1, 1), lambda i: (0, 0)),
            pl.BlockSpec((1, 1), lambda i: (0, 0)),
            pl.BlockSpec((1, 1), lambda i: (0, 0)),
            pl.BlockSpec((1, 1), lambda i: (0, 0)),
        ],
        out_shape=[
            jax.ShapeDtypeStruct((_N_ROWS,), jnp.int32),
            jax.ShapeDtypeStruct((_N_BLOCKS, _BLK, _CODE_DIM), jnp.float32),
            scal, scal, scal, scal, scal, scal,
        ],
        scratch_shapes=[pltpu.SMEM((1,), jnp.float32)],
        compiler_params=pltpu.CompilerParams(
            allow_input_fusion=[True, False, False]),
        interpret=interpret,
    )(flat3, w, u2)


_NW = 32
_B_PER_W = _N_ROWS // _NW
_CH = 512
_N_CH = _B_PER_W // _CH


_NB = 4


def _sc_gather_body(w_hbm, idx_hbm, out_hbm, idx_v, rows_v, *sems):
    gsems, wsems = sems[:_NB], sems[_NB:]
    wid = lax.axis_index("s") * 2 + lax.axis_index("c")
    base0 = wid * _B_PER_W
    pltpu.sync_copy(idx_hbm.at[pl.ds(base0, _B_PER_W)], idx_v)

    def _out_slice(c):
        g = base0 + c * _CH
        return out_hbm.at[g // 1024, pl.ds(g % 1024, _CH)]

    gops = [None] * _N_CH
    wops = [None] * _N_CH
    for c in range(_N_CH):
        b = c % _NB
        if c >= _NB:
            wops[c - _NB].wait()
        gops[c] = pltpu.async_copy(
            w_hbm.at[idx_v.at[pl.ds(c * _CH, _CH)]], rows_v.at[b], gsems[b])
        if c >= 1:
            gops[c - 1].wait()
            pb = (c - 1) % _NB
            wops[c - 1] = pltpu.async_copy(
                rows_v.at[pb], _out_slice(c - 1), wsems[pb])
    gops[_N_CH - 1].wait()
    lb = (_N_CH - 1) % _NB
    wops[_N_CH - 1] = pltpu.async_copy(
        rows_v.at[lb], _out_slice(_N_CH - 1), wsems[lb])
    for c in range(max(0, _N_CH - _NB), _N_CH):
        wops[c].wait()


@functools.cache
def _sc_gather():
    return functools.partial(
        pl.kernel,
        out_type=jax.ShapeDtypeStruct((128, 1024, _CODE_DIM), jnp.float32),
        mesh=plsc.VectorSubcoreMesh(core_axis_name="c", subcore_axis_name="s"),
        scratch_types=[
            pltpu.VMEM((_B_PER_W,), jnp.int32),
            pltpu.VMEM((_NB, _CH, _CODE_DIM), jnp.float32),
        ] + [pltpu.SemaphoreType.DMA] * (2 * _NB),
        compiler_params=pltpu.CompilerParams(use_tc_tiling_on_sc=False),
    )(_sc_gather_body)


_CH2 = 256
_NB2 = 4


def _sc_gather2_body(w_hbm, idx_hbm, out_hbm, idx_v, *sems):
    pl.run_scoped(
        functools.partial(_sc_gather2_inner, w_hbm, idx_hbm, out_hbm, idx_v,
                          sems),
        plsc.MemoryRef((_NB2, _CH2, 128), jnp.float32, pltpu.VMEM,
                       tiling=(8, 128)),
    )


def _sc_gather2_inner(w_hbm, idx_hbm, out_hbm, idx_v, sems, rows_v):
    gsems, wsems = sems[:_NB2], sems[_NB2:]
    wid = lax.axis_index("s") * 2 + lax.axis_index("c")
    base0 = wid * _B_PER_W
    pltpu.sync_copy(idx_hbm.at[pl.ds(base0, _B_PER_W)], idx_v)
    n_ch = _B_PER_W // _CH2

    def _out_slice(c):
        g = base0 + c * _CH2
        return out_hbm.at[g // 1024, pl.ds(g % 1024, _CH2)]

    gops = [None] * n_ch
    wops = [None] * n_ch
    for c in range(n_ch):
        b = c % _NB2
        if c >= _NB2:
            wops[c - _NB2].wait()
        gops[c] = pltpu.async_copy(
            w_hbm.at[idx_v.at[pl.ds(c * _CH2, _CH2)]], rows_v.at[b], gsems[b])
        if c >= 1:
            gops[c - 1].wait()
            pb = (c - 1) % _NB2
            wops[c - 1] = pltpu.async_copy(
                rows_v.at[pb, :, pl.ds(0, _CODE_DIM)], _out_slice(c - 1),
                wsems[pb])
    gops[n_ch - 1].wait()
    lb = (n_ch - 1) % _NB2
    wops[n_ch - 1] = pltpu.async_copy(
        rows_v.at[lb, :, pl.ds(0, _CODE_DIM)], _out_slice(n_ch - 1), wsems[lb])
    for c in range(max(0, n_ch - _NB2), n_ch):
        wops[c].wait()


@functools.cache
def _sc_gather2():
    return functools.partial(
        pl.kernel,
        out_type=jax.ShapeDtypeStruct((128, 1024, _CODE_DIM), jnp.float32),
        mesh=plsc.VectorSubcoreMesh(core_axis_name="c", subcore_axis_name="s"),
        scratch_types=[
            pltpu.VMEM((_B_PER_W,), jnp.int32),
        ] + [pltpu.SemaphoreType.DMA] * (2 * _NB2),
    )(_sc_gather2_body)


def kernel(inputs, W, usage_counts):
    flat3 = inputs.reshape(_N_BLOCKS, _BLK, _CODE_DIM)
    idx_flat, q3, tot, com, ent, var, dec, ue = _tc_search(
        flat3, W, usage_counts.reshape(1, _NUM_CODES))
    quantized = q3.reshape(inputs.shape)
    indices = idx_flat.reshape(inputs.shape[:-1])
    return (quantized, indices, tot.reshape(()), com.reshape(()),
            ent.reshape(()), var.reshape(()), dec.reshape(()),
            ue.reshape(()))


# speedup vs baseline: 4.1982x; 1.0003x over previous
import functools

import jax
import jax.numpy as jnp
import numpy as np
from jax import lax
from jax.experimental import pallas as pl
from jax.experimental.pallas import tpu as pltpu
from jax.experimental.pallas import tpu_sc as plsc

_NUM_CODES = 128
_CODE_DIM = 48
_COMMIT_W = 0.25
_EPS = 1e-05
_ENT_W = 0.1
_ENT_LO = 0.5
_ENT_HI = 0.9
_VAR_FLOOR = 0.05
_VAR_W = 0.001
_DECOR_W = 0.001

_N_ROWS = 128 * 1024
_BLK = 16384
_N_BLOCKS = _N_ROWS // _BLK


def _tc_body(x_ref, w_ref, u_ref, idx_ref, q_ref, tot_ref, com_ref, ent_ref,
             var_ref, dec_ref, ue_ref, acc_ref):
    i = pl.program_id(0)
    x = x_ref[...].reshape(_BLK, _CODE_DIM)
    w = w_ref[...]

    xsq = jnp.sum(x * x, axis=1, keepdims=True)
    wsq = jnp.sum(w * w, axis=1)
    mm = jax.lax.dot_general(x, w, (((1,), (1,)), ((), ())),
                             preferred_element_type=jnp.float32)
    d = xsq - 2.0 * mm + wsq[None, :]

    dt = d.T
    mind = jnp.min(dt, axis=0, keepdims=True)
    code_iota = lax.broadcasted_iota(jnp.int32, dt.shape, 0)
    idx = jnp.min(jnp.where(dt == mind, code_iota, _NUM_CODES), axis=0)
    idx_ref[...] = idx

    onehot_t = (code_iota == idx[None, :]).astype(jnp.float32)
    q = jax.lax.dot_general(onehot_t, w, (((0,), (0,)), ((), ())),
                            preferred_element_type=jnp.float32)
    q_ref[...] = q.reshape(q_ref.shape)

    blk_sum = jnp.sum(mind)

    @pl.when(i == 0)
    def _init():
        acc_ref[0] = blk_sum

    @pl.when(i > 0)
    def _acc():
        acc_ref[0] = acc_ref[0] + blk_sum

    @pl.when(i == _N_BLOCKS - 1)
    def _finalize():
        commit = _COMMIT_W * acc_ref[0] / float(_N_ROWS * _CODE_DIM)

        u = u_ref[...]
        p = u + _EPS
        p = p / jnp.maximum(jnp.sum(p), _EPS * _NUM_CODES)
        entropy = -jnp.sum(p * jnp.log(p + _EPS))
        ue = entropy / np.log(float(_NUM_CODES))
        gap = jnp.where(ue < _ENT_LO, _ENT_LO - ue,
                        jnp.where(ue > _ENT_HI, ue - _ENT_HI, 0.0))
        ent_loss = _ENT_W * gap * gap

        mean_w = jnp.mean(w, axis=0, keepdims=True)
        wc = w - mean_w
        variance = jnp.mean(wc * wc, axis=0, keepdims=True)
        var_loss = _VAR_W * jnp.mean(jnp.maximum(_VAR_FLOOR - variance, 0.0))

        cov = jax.lax.dot_general(wc, wc, (((0,), (0,)), ((), ())),
                                  preferred_element_type=jnp.float32)
        cov = cov / float(_NUM_CODES)
        ii = lax.broadcasted_iota(jnp.int32, cov.shape, 0)
        jj = lax.broadcasted_iota(jnp.int32, cov.shape, 1)
        off = jnp.where(ii == jj, 0.0, cov)
        dec_loss = _DECOR_W * jnp.sum(off * off) / float(_CODE_DIM * _CODE_DIM)

        tot_ref[...] = jnp.reshape(commit + ent_loss + var_loss + dec_loss,
                                   (1, 1))
        com_ref[...] = jnp.reshape(commit, (1, 1))
        ent_ref[...] = jnp.reshape(ent_loss, (1, 1))
        var_ref[...] = jnp.reshape(var_loss, (1, 1))
        dec_ref[...] = jnp.reshape(dec_loss, (1, 1))
        ue_ref[...] = jnp.reshape(ue, (1, 1))


def _tc_search(flat3, w, u2, interpret=False):
    scal = jax.ShapeDtypeStruct((1, 1), jnp.float32)
    return pl.pallas_call(
        _tc_body,
        grid=(_N_BLOCKS,),
        in_specs=[
            pl.BlockSpec((1, _BLK, _CODE_DIM), lambda i: (i, 0, 0)),
            pl.BlockSpec((_NUM_CODES, _CODE_DIM), lambda i: (0, 0)),
            pl.BlockSpec((1, _NUM_CODES), lambda i: (0, 0)),
        ],
        out_specs=[
            pl.BlockSpec((_BLK,), lambda i: (i,)),
            pl.BlockSpec((1, _BLK, _CODE_DIM), lambda i: (i, 0, 0)),
            pl.BlockSpec((1, 1), lambda i: (0, 0)),
            pl.BlockSpec((1, 1), lambda i: (0, 0)),
            pl.BlockSpec((1, 1), lambda i: (0, 0)),
            pl.BlockSpec((1, 1), lambda i: (0, 0)),
            pl.BlockSpec((1, 1), lambda i: (0, 0)),
            pl.BlockSpec((1, 1), lambda i: (0, 0)),
        ],
        out_shape=[
            jax.ShapeDtypeStruct((_N_ROWS,), jnp.int32),
            jax.ShapeDtypeStruct((_N_BLOCKS, _BLK, _CODE_DIM), jnp.float32),
            scal, scal, scal, scal, scal, scal,
        ],
        scratch_shapes=[pltpu.SMEM((1,), jnp.float32)],
        compiler_params=pltpu.CompilerParams(
            allow_input_fusion=[True, False, False]),
        interpret=interpret,
    )(flat3, w, u2)


_NW = 32
_B_PER_W = _N_ROWS // _NW
_CH = 512
_N_CH = _B_PER_W // _CH


_NB = 4


def _sc_gather_body(w_hbm, idx_hbm, out_hbm, idx_v, rows_v, *sems):
    gsems, wsems = sems[:_NB], sems[_NB:]
    wid = lax.axis_index("s") * 2 + lax.axis_index("c")
    base0 = wid * _B_PER_W
    pltpu.sync_copy(idx_hbm.at[pl.ds(base0, _B_PER_W)], idx_v)

    def _out_slice(c):
        g = base0 + c * _CH
        return out_hbm.at[g // 1024, pl.ds(g % 1024, _CH)]

    gops = [None] * _N_CH
    wops = [None] * _N_CH
    for c in range(_N_CH):
        b = c % _NB
        if c >= _NB:
            wops[c - _NB].wait()
        gops[c] = pltpu.async_copy(
            w_hbm.at[idx_v.at[pl.ds(c * _CH, _CH)]], rows_v.at[b], gsems[b])
        if c >= 1:
            gops[c - 1].wait()
            pb = (c - 1) % _NB
            wops[c - 1] = pltpu.async_copy(
                rows_v.at[pb], _out_slice(c - 1), wsems[pb])
    gops[_N_CH - 1].wait()
    lb = (_N_CH - 1) % _NB
    wops[_N_CH - 1] = pltpu.async_copy(
        rows_v.at[lb], _out_slice(_N_CH - 1), wsems[lb])
    for c in range(max(0, _N_CH - _NB), _N_CH):
        wops[c].wait()


@functools.cache
def _sc_gather():
    return functools.partial(
        pl.kernel,
        out_type=jax.ShapeDtypeStruct((128, 1024, _CODE_DIM), jnp.float32),
        mesh=plsc.VectorSubcoreMesh(core_axis_name="c", subcore_axis_name="s"),
        scratch_types=[
            pltpu.VMEM((_B_PER_W,), jnp.int32),
            pltpu.VMEM((_NB, _CH, _CODE_DIM), jnp.float32),
        ] + [pltpu.SemaphoreType.DMA] * (2 * _NB),
        compiler_params=pltpu.CompilerParams(use_tc_tiling_on_sc=False),
    )(_sc_gather_body)


_CH2 = 256
_NB2 = 4


def _sc_gather2_body(w_hbm, idx_hbm, out_hbm, idx_v, *sems):
    pl.run_scoped(
        functools.partial(_sc_gather2_inner, w_hbm, idx_hbm, out_hbm, idx_v,
                          sems),
        plsc.MemoryRef((_NB2, _CH2, 128), jnp.float32, pltpu.VMEM,
                       tiling=(8, 128)),
    )


def _sc_gather2_inner(w_hbm, idx_hbm, out_hbm, idx_v, sems, rows_v):
    gsems, wsems = sems[:_NB2], sems[_NB2:]
    wid = lax.axis_index("s") * 2 + lax.axis_index("c")
    base0 = wid * _B_PER_W
    pltpu.sync_copy(idx_hbm.at[pl.ds(base0, _B_PER_W)], idx_v)
    n_ch = _B_PER_W // _CH2

    def _out_slice(c):
        g = base0 + c * _CH2
        return out_hbm.at[g // 1024, pl.ds(g % 1024, _CH2)]

    gops = [None] * n_ch
    wops = [None] * n_ch
    for c in range(n_ch):
        b = c % _NB2
        if c >= _NB2:
            wops[c - _NB2].wait()
        gops[c] = pltpu.async_copy(
            w_hbm.at[idx_v.at[pl.ds(c * _CH2, _CH2)]], rows_v.at[b], gsems[b])
        if c >= 1:
            gops[c - 1].wait()
            pb = (c - 1) % _NB2
            wops[c - 1] = pltpu.async_copy(
                rows_v.at[pb, :, pl.ds(0, _CODE_DIM)], _out_slice(c - 1),
                wsems[pb])
    gops[n_ch - 1].wait()
    lb = (n_ch - 1) % _NB2
    wops[n_ch - 1] = pltpu.async_copy(
        rows_v.at[lb, :, pl.ds(0, _CODE_DIM)], _out_slice(n_ch - 1), wsems[lb])
    for c in range(max(0, n_ch - _NB2), n_ch):
        wops[c].wait()


@functools.cache
def _sc_gather2():
    return functools.partial(
        pl.kernel,
        out_type=jax.ShapeDtypeStruct((128, 1024, _CODE_DIM), jnp.float32),
        mesh=plsc.VectorSubcoreMesh(core_axis_name="c", subcore_axis_name="s"),
        scratch_types=[
            pltpu.VMEM((_B_PER_W,), jnp.int32),
        ] + [pltpu.SemaphoreType.DMA] * (2 * _NB2),
    )(_sc_gather2_body)


def kernel(inputs, W, usage_counts):
    flat3 = inputs.reshape(_N_BLOCKS, _BLK, _CODE_DIM)
    idx_flat, q3, tot, com, ent, var, dec, ue = _tc_search(
        flat3, W, usage_counts.reshape(1, _NUM_CODES))
    quantized = q3.reshape(inputs.shape)
    indices = idx_flat.reshape(inputs.shape[:-1])
    return (quantized, indices, tot.reshape(()), com.reshape(()),
            ent.reshape(()), var.reshape(()), dec.reshape(()),
            ue.reshape(()))
